# initial kernel scaffold (unmeasured)
import jax
import jax.numpy as jnp
from jax import lax
from jax.experimental import pallas as pl
from jax.experimental.pallas import tpu as pltpu

N_DEV = 8
N_HOPS = N_DEV - 1
CAPACITY = 25


def kernel(x, router_W, route_idx, expert_W):
    del router_W
    n_tok, _ = x.shape
    e_per, _, d_out = expert_W.shape
    n_exp = N_DEV * e_per

    idx = route_idx[:, 0]
    onehot = idx[:, None] == jnp.arange(n_exp, dtype=idx.dtype)[None, :]
    rank = jnp.cumsum(onehot.astype(jnp.int32), axis=0)
    kept = onehot & (rank <= CAPACITY)

    my = lax.axis_index("i")
    local_mask = lax.dynamic_slice(
        kept.astype(jnp.float32), (0, my * e_per), (n_tok, e_per)
    )
    xm = (local_mask.T[:, :, None] * x[None, :, :]).astype(jnp.bfloat16)
    w = expert_W.astype(jnp.bfloat16)

    def body(xm_ref, w_ref, out_ref, comm_ref, send_sems, recv_sems):
        my_pos = lax.axis_index("i")
        left = lax.rem(my_pos + N_DEV - 1, N_DEV)
        right = lax.rem(my_pos + 1, N_DEV)

        barrier_sem = pltpu.get_barrier_semaphore()
        for nbr in (left, right):
            pl.semaphore_signal(
                barrier_sem, inc=1,
                device_id=(nbr,), device_id_type=pl.DeviceIdType.MESH,
            )
        pl.semaphore_wait(barrier_sem, 2)

        partial = jnp.dot(
            xm_ref[0], w_ref[0], preferred_element_type=jnp.float32
        )
        for j in range(1, e_per):
            partial = partial + jnp.dot(
                xm_ref[j], w_ref[j], preferred_element_type=jnp.float32
            )
        out_ref[:, :] = partial
        comm_ref[0, :, :] = partial.astype(jnp.bfloat16)

        for h in range(N_HOPS):
            rdma = pltpu.make_async_remote_copy(
                src_ref=comm_ref.at[h],
                dst_ref=comm_ref.at[h + 1],
                send_sem=send_sems.at[h],
                recv_sem=recv_sems.at[h],
                device_id=(right,),
                device_id_type=pl.DeviceIdType.MESH,
            )
            rdma.start()
            rdma.wait()
            out_ref[:, :] += comm_ref[h + 1, :, :].astype(jnp.float32)

        @pl.run_scoped
        def _(second_barrier=pltpu.SemaphoreType.REGULAR):
            for nbr in (left, right):
                pl.semaphore_signal(
                    second_barrier, inc=1,
                    device_id=(nbr,), device_id_type=pl.DeviceIdType.MESH,
                )
            pl.semaphore_wait(second_barrier, 2)

    return pl.pallas_call(
        body,
        out_shape=jax.ShapeDtypeStruct((n_tok, d_out), jnp.float32),
        in_specs=[
            pl.BlockSpec(memory_space=pltpu.VMEM),
            pl.BlockSpec(memory_space=pltpu.VMEM),
        ],
        out_specs=pl.BlockSpec(memory_space=pltpu.VMEM),
        scratch_shapes=[
            pltpu.VMEM((N_DEV, n_tok, d_out), jnp.bfloat16),
            pltpu.SemaphoreType.DMA((N_HOPS,)),
            pltpu.SemaphoreType.DMA((N_HOPS,)),
        ],
        compiler_params=pltpu.CompilerParams(collective_id=0),
    )(xm, w)


# baseline (device time: 63121 ns/iter reference)
import functools

import jax
import jax.numpy as jnp
from jax import lax
from jax.experimental import pallas as pl
from jax.experimental.pallas import tpu as pltpu

N_DEV = 8
N_HOPS = N_DEV - 1
CAPACITY = 25


def kernel(x, router_W, route_idx, expert_W):
    del router_W
    n_tok, _ = x.shape
    e_per, _, d_out = expert_W.shape
    n_exp = N_DEV * e_per

    idx = route_idx[:, 0]
    onehot = idx[:, None] == jnp.arange(n_exp, dtype=idx.dtype)[None, :]
    rank = jnp.cumsum(onehot.astype(jnp.int32), axis=0)
    kept = onehot & (rank <= CAPACITY)

    my = lax.axis_index("i")
    local_mask = lax.dynamic_slice(
        kept.astype(jnp.float32), (0, my * e_per), (n_tok, e_per)
    )
    xm = (local_mask.T[:, :, None] * x[None, :, :]).astype(jnp.bfloat16)
    w = expert_W.astype(jnp.bfloat16)

    def body(xm_ref, w_ref, out_ref, comm_ref, send_sems, recv_sems):
        my_pos = lax.axis_index("i")
        left = lax.rem(my_pos + N_DEV - 1, N_DEV)
        right = lax.rem(my_pos + 1, N_DEV)

        barrier_sem = pltpu.get_barrier_semaphore()
        for nbr in (left, right):
            pl.semaphore_signal(
                barrier_sem, inc=1,
                device_id=(nbr,), device_id_type=pl.DeviceIdType.MESH,
            )
        pl.semaphore_wait(barrier_sem, 2)

        partial = jnp.dot(
            xm_ref[0], w_ref[0], preferred_element_type=jnp.float32
        )
        for j in range(1, e_per):
            partial = partial + jnp.dot(
                xm_ref[j], w_ref[j], preferred_element_type=jnp.float32
            )
        out_ref[:, :] = partial
        comm_ref[0, :, :] = partial.astype(jnp.bfloat16)

        for h in range(N_HOPS):
            rdma = pltpu.make_async_remote_copy(
                src_ref=comm_ref.at[h],
                dst_ref=comm_ref.at[h + 1],
                send_sem=send_sems.at[h],
                recv_sem=recv_sems.at[h],
                device_id=(right,),
                device_id_type=pl.DeviceIdType.MESH,
            )
            rdma.start()
            rdma.wait()
            out_ref[:, :] += comm_ref[h + 1, :, :].astype(jnp.float32)

        @functools.partial(
            pl.run_scoped, second_barrier=pltpu.SemaphoreType.REGULAR
        )
        def _(second_barrier):
            for nbr in (left, right):
                pl.semaphore_signal(
                    second_barrier, inc=1,
                    device_id=(nbr,), device_id_type=pl.DeviceIdType.MESH,
                )
            pl.semaphore_wait(second_barrier, 2)

    return pl.pallas_call(
        body,
        out_shape=jax.ShapeDtypeStruct((n_tok, d_out), jnp.float32),
        in_specs=[
            pl.BlockSpec(memory_space=pltpu.VMEM),
            pl.BlockSpec(memory_space=pltpu.VMEM),
        ],
        out_specs=pl.BlockSpec(memory_space=pltpu.VMEM),
        scratch_shapes=[
            pltpu.VMEM((N_DEV, n_tok, d_out), jnp.bfloat16),
            pltpu.SemaphoreType.DMA((N_HOPS,)),
            pltpu.SemaphoreType.DMA((N_HOPS,)),
        ],
        compiler_params=pltpu.CompilerParams(collective_id=0),
    )(xm, w)


# device time: 26605 ns/iter; 2.3725x vs baseline; 2.3725x over previous
import functools

import jax
import jax.numpy as jnp
from jax import lax
from jax.experimental import pallas as pl
from jax.experimental.pallas import tpu as pltpu

N_DEV = 8
R_HOPS = N_DEV // 2
L_HOPS = N_DEV - 1 - R_HOPS
CAPACITY = 25
PAD = 64


def kernel(x, router_W, route_idx, expert_W):
    del router_W
    n_tok, _ = x.shape
    e_per, _, d_out = expert_W.shape
    n_exp = N_DEV * e_per

    idx = route_idx[:, 0]
    onehot = idx[:, None] == jnp.arange(n_exp, dtype=idx.dtype)[None, :]
    rank = jnp.cumsum(onehot.astype(jnp.int32), axis=0)
    kept = onehot & (rank <= CAPACITY)
    kept_tok = jnp.any(kept, axis=1)
    owner = idx // e_per
    loc = idx % e_per

    dev_oh = (owner[:, None] == jnp.arange(N_DEV, dtype=idx.dtype)) & (
        kept_tok[:, None]
    )
    rank_dev = jnp.cumsum(dev_oh.astype(jnp.int32), axis=0) - 1
    r = jnp.take_along_axis(rank_dev, owner[:, None].astype(jnp.int32), axis=1)[
        :, 0
    ]

    slot = owner.astype(jnp.int32) * PAD + r
    P = (
        (jnp.arange(N_DEV * PAD, dtype=jnp.int32)[None, :] == slot[:, None])
        & kept_tok[:, None]
    ).astype(jnp.bfloat16)

    my = lax.axis_index("i")
    my_tok = kept_tok & (owner == my)
    sfull = (jnp.arange(PAD, dtype=jnp.int32)[:, None] == r[None, :]) & (
        my_tok[None, :]
    )
    S = (
        sfull[None, :, :]
        & (loc[None, None, :] == jnp.arange(e_per, dtype=idx.dtype)[:, None, None])
    ).astype(jnp.bfloat16)

    xb = x.astype(jnp.bfloat16)
    w = expert_W.astype(jnp.bfloat16)

    def body(s_ref, x_ref, w_ref, p_ref, out_ref, gath_ref,
             r_send, r_recv, l_send, l_recv):
        my_pos = lax.axis_index("i")
        left = lax.rem(my_pos + N_DEV - 1, N_DEV)
        right = lax.rem(my_pos + 1, N_DEV)

        barrier_sem = pltpu.get_barrier_semaphore()
        for nbr in (left, right):
            pl.semaphore_signal(
                barrier_sem, inc=1,
                device_id=(nbr,), device_id_type=pl.DeviceIdType.MESH,
            )
        pl.semaphore_wait(barrier_sem, 2)

        c = jnp.dot(
            jnp.dot(s_ref[0], x_ref[:, :], preferred_element_type=jnp.float32
                    ).astype(jnp.bfloat16),
            w_ref[0], preferred_element_type=jnp.float32,
        )
        for j in range(1, e_per):
            c = c + jnp.dot(
                jnp.dot(s_ref[j], x_ref[:, :],
                        preferred_element_type=jnp.float32
                        ).astype(jnp.bfloat16),
                w_ref[j], preferred_element_type=jnp.float32,
            )
        gath_ref[pl.ds(my_pos * PAD, PAD), :] = c.astype(jnp.bfloat16)

        for h in range(R_HOPS):
            r_org = lax.rem(my_pos - h + N_DEV, N_DEV)
            rdma_r = pltpu.make_async_remote_copy(
                src_ref=gath_ref.at[pl.ds(r_org * PAD, PAD), :],
                dst_ref=gath_ref.at[pl.ds(r_org * PAD, PAD), :],
                send_sem=r_send.at[h],
                recv_sem=r_recv.at[h],
                device_id=(right,),
                device_id_type=pl.DeviceIdType.MESH,
            )
            rdma_r.start()
            if h < L_HOPS:
                l_org = lax.rem(my_pos + h, N_DEV)
                rdma_l = pltpu.make_async_remote_copy(
                    src_ref=gath_ref.at[pl.ds(l_org * PAD, PAD), :],
                    dst_ref=gath_ref.at[pl.ds(l_org * PAD, PAD), :],
                    send_sem=l_send.at[h],
                    recv_sem=l_recv.at[h],
                    device_id=(left,),
                    device_id_type=pl.DeviceIdType.MESH,
                )
                rdma_l.start()
                rdma_l.wait()
            rdma_r.wait()

        out_ref[:, :] = jnp.dot(
            p_ref[:, :], gath_ref[:, :], preferred_element_type=jnp.float32
        )

        @functools.partial(
            pl.run_scoped, second_barrier=pltpu.SemaphoreType.REGULAR
        )
        def _(second_barrier):
            for nbr in (left, right):
                pl.semaphore_signal(
                    second_barrier, inc=1,
                    device_id=(nbr,), device_id_type=pl.DeviceIdType.MESH,
                )
            pl.semaphore_wait(second_barrier, 2)

    return pl.pallas_call(
        body,
        out_shape=jax.ShapeDtypeStruct((n_tok, d_out), jnp.float32),
        in_specs=[
            pl.BlockSpec(memory_space=pltpu.VMEM),
            pl.BlockSpec(memory_space=pltpu.VMEM),
            pl.BlockSpec(memory_space=pltpu.VMEM),
            pl.BlockSpec(memory_space=pltpu.VMEM),
        ],
        out_specs=pl.BlockSpec(memory_space=pltpu.VMEM),
        scratch_shapes=[
            pltpu.VMEM((N_DEV * PAD, d_out), jnp.bfloat16),
            pltpu.SemaphoreType.DMA((R_HOPS,)),
            pltpu.SemaphoreType.DMA((R_HOPS,)),
            pltpu.SemaphoreType.DMA((L_HOPS,)),
            pltpu.SemaphoreType.DMA((L_HOPS,)),
        ],
        compiler_params=pltpu.CompilerParams(collective_id=0),
    )(S, xb, w, P)


# device time: 12762 ns/iter; 4.9460x vs baseline; 2.0847x over previous
import jax
import jax.numpy as jnp
from jax import lax
from jax.experimental import pallas as pl
from jax.experimental.pallas import tpu as pltpu

N_DEV = 8
N_PEERS = N_DEV - 1
CAPACITY = 25
PAD = 64


def kernel(x, router_W, route_idx, expert_W):
    del router_W
    n_tok, _ = x.shape
    e_per, _, d_out = expert_W.shape
    n_exp = N_DEV * e_per

    idx_col = route_idx.astype(jnp.int32)
    idx_row = idx_col.reshape(1, n_tok)
    xb = x.astype(jnp.bfloat16)
    w = expert_W.astype(jnp.bfloat16)

    def body(ic_ref, ir_ref, x_ref, w_ref, out_ref, gath_ref,
             send_sems, recv_sems):
        my_pos = lax.axis_index("i")
        f32, bf16, i32 = jnp.float32, jnp.bfloat16, jnp.int32

        def iota(shape, dim):
            return lax.broadcasted_iota(i32, shape, dim)

        ir = ir_ref[:, :]
        oh_r = iota((n_exp, n_tok), 0) == ir
        u_incl = (iota((n_tok, n_tok), 0) <= iota((n_tok, n_tok), 1)
                  ).astype(bf16)
        rank_r = jnp.dot(oh_r.astype(bf16), u_incl,
                         preferred_element_type=f32)
        kept_r = oh_r & (rank_r <= CAPACITY)
        kept_tok_r = jnp.sum(kept_r.astype(f32), axis=0, keepdims=True) > 0
        owner_r = ir // e_per
        loc_r = lax.rem(ir, e_per)
        dev_oh_r = (iota((N_DEV, n_tok), 0) == owner_r) & kept_tok_r
        u_excl = (iota((n_tok, n_tok), 0) < iota((n_tok, n_tok), 1)
                  ).astype(bf16)
        rdev_r = jnp.dot(dev_oh_r.astype(bf16), u_excl,
                         preferred_element_type=f32)
        r_row = jnp.sum(rdev_r * dev_oh_r.astype(f32), axis=0,
                        keepdims=True)
        my_tok_r = kept_tok_r & (owner_r == my_pos)

        rows_iota = iota((PAD, n_tok), 0).astype(f32)
        c = None
        for j in range(e_per):
            sj = ((rows_iota == r_row) & my_tok_r & (loc_r == j)
                  ).astype(bf16)
            cj = jnp.dot(
                jnp.dot(sj, x_ref[:, :], preferred_element_type=f32
                        ).astype(bf16),
                w_ref[j], preferred_element_type=f32,
            )
            c = cj if c is None else c + cj
        gath_ref[pl.ds(my_pos * PAD, PAD), :] = c.astype(bf16)

        barrier_sem = pltpu.get_barrier_semaphore()
        for k in range(1, N_DEV):
            pl.semaphore_signal(
                barrier_sem, inc=1,
                device_id=(lax.rem(my_pos + k, N_DEV),),
                device_id_type=pl.DeviceIdType.MESH,
            )
        pl.semaphore_wait(barrier_sem, N_PEERS)

        rdmas = []
        for k in range(1, N_DEV):
            rdma = pltpu.make_async_remote_copy(
                src_ref=gath_ref.at[pl.ds(my_pos * PAD, PAD), :],
                dst_ref=gath_ref.at[pl.ds(my_pos * PAD, PAD), :],
                send_sem=send_sems.at[k - 1],
                recv_sem=recv_sems.at[k - 1],
                device_id=(lax.rem(my_pos + k, N_DEV),),
                device_id_type=pl.DeviceIdType.MESH,
            )
            rdma.start()
            rdmas.append(rdma)

        ic = ic_ref[:, :]
        oh_c = ic == iota((n_tok, n_exp), 1)
        l_incl = (iota((n_tok, n_tok), 0) >= iota((n_tok, n_tok), 1)
                  ).astype(bf16)
        rank_c = jnp.dot(l_incl, oh_c.astype(bf16),
                         preferred_element_type=f32)
        kept_c = oh_c & (rank_c <= CAPACITY)
        kept_tok_c = jnp.sum(kept_c.astype(f32), axis=1, keepdims=True) > 0
        owner_c = ic // e_per
        dev_oh_c = (iota((n_tok, N_DEV), 1) == owner_c) & kept_tok_c
        l_excl = (iota((n_tok, n_tok), 0) > iota((n_tok, n_tok), 1)
                  ).astype(bf16)
        rdev_c = jnp.dot(l_excl, dev_oh_c.astype(bf16),
                         preferred_element_type=f32)
        r_col = jnp.sum(rdev_c * dev_oh_c.astype(f32), axis=1,
                        keepdims=True)
        slot_c = owner_c.astype(f32) * PAD + r_col
        p = ((iota((n_tok, N_DEV * PAD), 1).astype(f32) == slot_c)
             & kept_tok_c).astype(bf16)

        for rdma in rdmas:
            rdma.wait()

        out_ref[:, :] = jnp.dot(p, gath_ref[:, :],
                                preferred_element_type=f32)

    return pl.pallas_call(
        body,
        out_shape=jax.ShapeDtypeStruct((n_tok, d_out), jnp.float32),
        in_specs=[
            pl.BlockSpec(memory_space=pltpu.VMEM),
            pl.BlockSpec(memory_space=pltpu.VMEM),
            pl.BlockSpec(memory_space=pltpu.VMEM),
            pl.BlockSpec(memory_space=pltpu.VMEM),
        ],
        out_specs=pl.BlockSpec(memory_space=pltpu.VMEM),
        scratch_shapes=[
            pltpu.VMEM((N_DEV * PAD, d_out), jnp.bfloat16),
            pltpu.SemaphoreType.DMA((N_PEERS,)),
            pltpu.SemaphoreType.DMA((N_PEERS,)),
        ],
        compiler_params=pltpu.CompilerParams(collective_id=0),
    )(idx_col, idx_row, xb, w)
